# baseline (speedup 1.0000x reference)
import jax
import jax.numpy as jnp
from jax import lax
from jax.experimental import pallas as pl
from jax.experimental.pallas import tpu as pltpu

_NBUF = 8
_B, _G, _H, _W, _D = 16, 9, 30, 30, 256
_N = _B * _G


def _body(x_ref, row_ref, col_ref, io_ref, pair_ref, out_ref,
          comb_ref, inbuf, outbuf, insem, outsem):
    for i in range(_NBUF):
        pltpu.make_async_copy(x_ref.at[i], inbuf.at[i], insem.at[i]).start()

    d4 = row_ref.shape[1]
    row_b = lax.broadcast_in_dim(row_ref[...], (_H, _W, d4), (0, 2))
    col_b = lax.broadcast_in_dim(col_ref[...], (_H, _W, d4), (1, 2))
    for g in range(_G):
        io_b = lax.broadcast_in_dim(io_ref[pl.ds(g % 2, 1), :], (_H, _W, d4), (1, 2))
        pair_b = lax.broadcast_in_dim(pair_ref[pl.ds(g // 2, 1), :], (_H, _W, d4), (1, 2))
        comb_ref[g] = jnp.concatenate([row_b, col_b, io_b, pair_b], axis=-1)

    for i in range(_N):
        slot = i % _NBUF
        pltpu.make_async_copy(x_ref.at[i], inbuf.at[slot], insem.at[slot]).wait()
        if i >= _NBUF:
            pltpu.make_async_copy(
                outbuf.at[slot], out_ref.at[i - _NBUF], outsem.at[slot]).wait()
        outbuf[slot] = inbuf[slot] + comb_ref[i % _G]
        pltpu.make_async_copy(outbuf.at[slot], out_ref.at[i], outsem.at[slot]).start()
        nxt = i + _NBUF
        if nxt < _N:
            pltpu.make_async_copy(x_ref.at[nxt], inbuf.at[slot], insem.at[slot]).start()

    for i in range(_N - _NBUF, _N):
        slot = i % _NBUF
        pltpu.make_async_copy(outbuf.at[slot], out_ref.at[i], outsem.at[slot]).wait()


def kernel(x, row_table, col_table, io_table, pair_table):
    B, G, H, W, D = x.shape
    xf = x.reshape(B * G, H, W, D)
    hbm = pl.BlockSpec(memory_space=pltpu.MemorySpace.HBM)
    vmem = pl.BlockSpec(memory_space=pltpu.MemorySpace.VMEM)
    out = pl.pallas_call(
        _body,
        in_specs=[hbm, vmem, vmem, vmem, vmem],
        out_specs=hbm,
        out_shape=jax.ShapeDtypeStruct((B * G, H, W, D), x.dtype),
        scratch_shapes=[
            pltpu.VMEM((G, H, W, D), jnp.float32),
            pltpu.VMEM((_NBUF, H, W, D), jnp.float32),
            pltpu.VMEM((_NBUF, H, W, D), jnp.float32),
            pltpu.SemaphoreType.DMA((_NBUF,)),
            pltpu.SemaphoreType.DMA((_NBUF,)),
        ],
    )(xf, row_table, col_table, io_table, pair_table)
    return out.reshape(B, G, H, W, D)
